# baseline (speedup 1.0000x reference)
import jax
import jax.numpy as jnp
from jax import lax
from jax.experimental import pallas as pl
from jax.experimental.pallas import tpu as pltpu
from jax.experimental.pallas import tpu_sc as plsc

_VOCAB = 1000000
_EMBED = 64
_MAXLEN = 200
_BATCH = 4096

_L = 16
_NW = 32
_EC = _EMBED // _L
_SEQ_PER_W = _BATCH // _NW
_CS = 2
_ROWS = _CS * _MAXLEN
_NCH = _SEQ_PER_W // _CS
_GSPLITS = ((0, 128), (128, 128), (256, 128), (384, 16))
_IDX_VREGS = _ROWS // _L


def _emb_body(seq_hbm, tok_hbm, pos_hbm, out_hbm, idx_v, rows_v, pos_v, sem):
    wid = lax.axis_index("s") * 2 + lax.axis_index("c")
    row0 = wid * (_SEQ_PER_W * _MAXLEN)

    pltpu.sync_copy(pos_hbm, pos_v)

    @pl.loop(0, _NCH)
    def _chunk(g):
        base = row0 + g * _ROWS
        pltpu.sync_copy(seq_hbm.at[pl.ds(base, _ROWS)], idx_v)

        cps = [
            pltpu.async_copy(
                tok_hbm.at[idx_v.at[pl.ds(off, num)]],
                rows_v.at[pl.ds(off, num)],
                sem,
            )
            for off, num in _GSPLITS
        ]
        for cp in cps:
            cp.wait()

        @pl.loop(0, _IDX_VREGS)
        def _fix(m):
            v = idx_v[pl.ds(m * _L, _L)]
            mask = v == 0
            nzero = jnp.sum(jnp.where(mask, 1, 0))

            @pl.when(nzero > 0)
            def _():
                rows = lax.iota(jnp.int32, _L)
                zeros = jnp.zeros((_L,), jnp.float32)
                tile = rows_v.at[pl.ds(m * _L, _L), :]
                for col in range(_EMBED):
                    plsc.store_scatter(
                        tile,
                        [rows, jnp.full((_L,), col, jnp.int32)],
                        zeros,
                        mask=mask,
                    )

        @pl.loop(0, _MAXLEN)
        def _add(l):
            for s in range(_CS):
                r = s * _MAXLEN + l
                for c in range(_EC):
                    plsc.addupdate(
                        rows_v.at[r, pl.ds(c * _L, _L)],
                        pos_v[l, pl.ds(c * _L, _L)],
                    )

        pltpu.sync_copy(rows_v, out_hbm.at[pl.ds(base, _ROWS)])


@jax.jit
def _emb_call(seq_flat, tok_table, pos_table):
    return pl.kernel(
        _emb_body,
        out_type=jax.ShapeDtypeStruct((_BATCH * _MAXLEN, _EMBED), jnp.float32),
        mesh=plsc.VectorSubcoreMesh(core_axis_name="c", subcore_axis_name="s"),
        compiler_params=pltpu.CompilerParams(
            use_tc_tiling_on_sc=False, needs_layout_passes=False
        ),
        scratch_types=[
            pltpu.VMEM((_ROWS,), jnp.int32),
            pltpu.VMEM((_ROWS, _EMBED), jnp.float32),
            pltpu.VMEM((_MAXLEN, _EMBED), jnp.float32),
            pltpu.SemaphoreType.DMA,
        ],
    )(seq_flat, tok_table, pos_table)


def kernel(sequence, token_table, pos_table):
    seq_flat = sequence.reshape(_BATCH * _MAXLEN)
    out = _emb_call(seq_flat, token_table, pos_table)
    return out.reshape(_BATCH, _MAXLEN, _EMBED)
